# baseline (speedup 1.0000x reference)
import functools

import jax
import jax.numpy as jnp
from jax import lax
from jax.experimental import pallas as pl
from jax.experimental.pallas import tpu as pltpu
from jax.experimental.pallas import tpu_sc as plsc

LATENT = 32
LANES = 16
TILE_W = 128
SUBL = 8


KT = 512


@functools.lru_cache(maxsize=None)
def _build_reformat(R):
    n_grp = LATENT // SUBL
    tpg = (R + TILE_W - 1) // TILE_W
    n_j = (tpg + KT - 1) // KT

    def body(inu_ref, ini_ref, outu_ref, outi_ref):
        for k in range(KT):
            csl = slice(k * TILE_W, (k + 1) * TILE_W)
            outu_ref[0, k] = inu_ref[0, :, csl]
            outi_ref[0, k] = ini_ref[0, :, csl]

    in_spec = pl.BlockSpec((1, SUBL, KT * TILE_W), lambda g, j: (g, 0, j))
    out_spec = pl.BlockSpec((1, KT, SUBL, TILE_W), lambda g, j: (g, j, 0, 0))
    oshape = jax.ShapeDtypeStruct((n_grp, tpg, SUBL, TILE_W), jnp.float32)
    return pl.pallas_call(
        body,
        grid=(n_grp, n_j),
        in_specs=[in_spec, in_spec],
        out_specs=[out_spec, out_spec],
        out_shape=(oshape, oshape),
    )


@functools.lru_cache(maxsize=None)
def _build_gather(B, R):
    info = plsc.get_sparse_core_info()
    nc, ns = info.num_cores, info.num_subcores
    nw = nc * ns
    assert B % (nw * LANES) == 0
    b_per_w = B // nw
    tpg = (R + TILE_W - 1) // TILE_W
    wpg = tpg * SUBL * TILE_W

    mesh = plsc.VectorSubcoreMesh(core_axis_name="c", subcore_axis_name="s")

    @functools.partial(
        pl.kernel,
        mesh=mesh,
        out_type=jax.ShapeDtypeStruct((B,), jnp.float32),
        compiler_params=pltpu.CompilerParams(needs_layout_passes=False),
        scratch_types=[
            pltpu.VMEM((b_per_w,), jnp.int32),
            pltpu.VMEM((b_per_w,), jnp.int32),
            pltpu.VMEM((b_per_w,), jnp.int32),
            pltpu.VMEM((b_per_w,), jnp.int32),
            pltpu.VMEM((LATENT * b_per_w,), jnp.float32),
            pltpu.VMEM((LATENT * b_per_w,), jnp.float32),
            pltpu.VMEM(((LATENT + 1) * LANES,), jnp.float32),
            pltpu.VMEM((b_per_w,), jnp.float32),
            pltpu.SemaphoreType.DMA,
            pltpu.SemaphoreType.DMA,
        ],
    )
    def gmf(user_hbm, item_hbm, ut_hbm, it_hbm, wb_hbm, out_hbm,
            uidx_v, iidx_v, urb_v, irb_v, ubuf_v, ibuf_v, wb_v, out_v,
            sem_u, sem_i):
        wid = lax.axis_index("s") * nc + lax.axis_index("c")
        base = wid * b_per_w

        pltpu.sync_copy(user_hbm.at[pl.ds(base, b_per_w)], uidx_v)
        pltpu.sync_copy(item_hbm.at[pl.ds(base, b_per_w)], iidx_v)
        pltpu.sync_copy(wb_hbm, wb_v)

        def prep(k, carry):
            sl = pl.ds(k * LANES, LANES)
            for idx_v, rb_v in ((uidx_v, urb_v), (iidx_v, irb_v)):
                r = idx_v[sl]
                rb_v[sl] = (
                    jnp.left_shift(jnp.right_shift(r, 7), 10)
                    + jnp.bitwise_and(r, TILE_W - 1))
            return carry

        lax.fori_loop(0, b_per_w // LANES, prep, 0)

        def fire(c, carry):
            off = (jnp.right_shift(c, 3) * wpg
                   + jnp.bitwise_and(c, SUBL - 1) * TILE_W)
            cb = c * b_per_w
            for k in range(b_per_w // LANES):
                ksl = pl.ds(k * LANES, LANES)
                dsl = pl.ds(cb + k * LANES, LANES)
                pltpu.async_copy(
                    ut_hbm.at[urb_v[ksl] + off], ubuf_v.at[dsl], sem_u)
                pltpu.async_copy(
                    it_hbm.at[irb_v[ksl] + off], ibuf_v.at[dsl], sem_i)
            return carry

        lax.fori_loop(0, LATENT, fire, 0)

        pltpu.make_async_copy(
            ut_hbm.at[pl.ds(0, LATENT * b_per_w)], ubuf_v, sem_u).wait()
        pltpu.make_async_copy(
            it_hbm.at[pl.ds(0, LATENT * b_per_w)], ibuf_v, sem_i).wait()

        bvec = wb_v[pl.ds(LATENT * LANES, LANES)]

        def comp(t, carry):
            sl = pl.ds(t * LANES, LANES)
            acc = bvec
            for c in range(LATENT):
                wc = wb_v[pl.ds(c * LANES, LANES)]
                csl = pl.ds(c * b_per_w + t * LANES, LANES)
                acc = acc + wc * ubuf_v[csl] * ibuf_v[csl]
            out_v[sl] = acc
            return carry

        lax.fori_loop(0, b_per_w // LANES, comp, 0)

        pltpu.sync_copy(out_v, out_hbm.at[pl.ds(base, b_per_w)])

    return gmf


def kernel(user, item, u_table, i_table, W, b):
    B = user.shape[0]
    R = u_table.shape[0]
    user1d = user.reshape(B)
    item1d = item.reshape(B)
    wfull = jnp.concatenate([
        jnp.broadcast_to(W.reshape(LATENT, 1), (LATENT, LANES)).reshape(-1),
        jnp.broadcast_to(b.reshape(1), (LANES,)),
    ])
    n_grp = LATENT // SUBL
    ut3 = u_table.T.reshape(n_grp, SUBL, R)
    it3 = i_table.T.reshape(n_grp, SUBL, R)
    uflat, iflat = _build_reformat(R)(ut3, it3)
    out = _build_gather(B, R)(
        user1d, item1d, uflat.reshape(-1), iflat.reshape(-1), wfull)
    return out.reshape(B, 1)


# speedup vs baseline: 1.0143x; 1.0143x over previous
import functools

import jax
import jax.numpy as jnp
from jax import lax
from jax.experimental import pallas as pl
from jax.experimental.pallas import tpu as pltpu
from jax.experimental.pallas import tpu_sc as plsc

LATENT = 32
LANES = 16
TILE_W = 128
SUBL = 8


KT = 1024


@functools.lru_cache(maxsize=None)
def _build_reformat(R):
    n_grp = LATENT // SUBL
    tpg = (R + TILE_W - 1) // TILE_W
    n_j = (tpg + KT - 1) // KT

    def body(inu_ref, ini_ref, outu_ref, outi_ref):
        for k in range(KT):
            csl = slice(k * TILE_W, (k + 1) * TILE_W)
            outu_ref[0, k] = inu_ref[0, :, csl]
            outi_ref[0, k] = ini_ref[0, :, csl]

    in_spec = pl.BlockSpec((1, SUBL, KT * TILE_W), lambda g, j: (g, 0, j))
    out_spec = pl.BlockSpec((1, KT, SUBL, TILE_W), lambda g, j: (g, j, 0, 0))
    oshape = jax.ShapeDtypeStruct((n_grp, tpg, SUBL, TILE_W), jnp.float32)
    return pl.pallas_call(
        body,
        grid=(n_grp, n_j),
        in_specs=[in_spec, in_spec],
        out_specs=[out_spec, out_spec],
        out_shape=(oshape, oshape),
    )


@functools.lru_cache(maxsize=None)
def _build_gather(B, R):
    info = plsc.get_sparse_core_info()
    nc, ns = info.num_cores, info.num_subcores
    nw = nc * ns
    assert B % (nw * LANES) == 0
    b_per_w = B // nw
    tpg = (R + TILE_W - 1) // TILE_W
    wpg = tpg * SUBL * TILE_W

    mesh = plsc.VectorSubcoreMesh(core_axis_name="c", subcore_axis_name="s")

    @functools.partial(
        pl.kernel,
        mesh=mesh,
        out_type=jax.ShapeDtypeStruct((B,), jnp.float32),
        compiler_params=pltpu.CompilerParams(needs_layout_passes=False),
        scratch_types=[
            pltpu.VMEM((b_per_w,), jnp.int32),
            pltpu.VMEM((b_per_w,), jnp.int32),
            pltpu.VMEM((b_per_w,), jnp.int32),
            pltpu.VMEM((b_per_w,), jnp.int32),
            pltpu.VMEM((LATENT * b_per_w,), jnp.float32),
            pltpu.VMEM((LATENT * b_per_w,), jnp.float32),
            pltpu.VMEM(((LATENT + 1) * LANES,), jnp.float32),
            pltpu.VMEM((b_per_w,), jnp.float32),
            pltpu.SemaphoreType.DMA,
            pltpu.SemaphoreType.DMA,
        ],
    )
    def gmf(user_hbm, item_hbm, ut_hbm, it_hbm, wb_hbm, out_hbm,
            uidx_v, iidx_v, urb_v, irb_v, ubuf_v, ibuf_v, wb_v, out_v,
            sem_u, sem_i):
        wid = lax.axis_index("s") * nc + lax.axis_index("c")
        base = wid * b_per_w

        pltpu.sync_copy(user_hbm.at[pl.ds(base, b_per_w)], uidx_v)
        pltpu.sync_copy(item_hbm.at[pl.ds(base, b_per_w)], iidx_v)
        pltpu.sync_copy(wb_hbm, wb_v)

        def prep(k, carry):
            sl = pl.ds(k * LANES, LANES)
            for idx_v, rb_v in ((uidx_v, urb_v), (iidx_v, irb_v)):
                r = idx_v[sl]
                rb_v[sl] = (
                    jnp.left_shift(jnp.right_shift(r, 7), 10)
                    + jnp.bitwise_and(r, TILE_W - 1))
            return carry

        lax.fori_loop(0, b_per_w // LANES, prep, 0)

        def fire(c, carry):
            off = (jnp.right_shift(c, 3) * wpg
                   + jnp.bitwise_and(c, SUBL - 1) * TILE_W)
            cb = c * b_per_w
            for k in range(b_per_w // LANES):
                ksl = pl.ds(k * LANES, LANES)
                dsl = pl.ds(cb + k * LANES, LANES)
                pltpu.async_copy(
                    ut_hbm.at[urb_v[ksl] + off], ubuf_v.at[dsl], sem_u)
                pltpu.async_copy(
                    it_hbm.at[irb_v[ksl] + off], ibuf_v.at[dsl], sem_i)
            return carry

        lax.fori_loop(0, LATENT, fire, 0)

        pltpu.make_async_copy(
            ut_hbm.at[pl.ds(0, LATENT * b_per_w)], ubuf_v, sem_u).wait()
        pltpu.make_async_copy(
            it_hbm.at[pl.ds(0, LATENT * b_per_w)], ibuf_v, sem_i).wait()

        bvec = wb_v[pl.ds(LATENT * LANES, LANES)]

        def comp(t, carry):
            sl = pl.ds(t * LANES, LANES)
            acc = bvec
            for c in range(LATENT):
                wc = wb_v[pl.ds(c * LANES, LANES)]
                csl = pl.ds(c * b_per_w + t * LANES, LANES)
                acc = acc + wc * ubuf_v[csl] * ibuf_v[csl]
            out_v[sl] = acc
            return carry

        lax.fori_loop(0, b_per_w // LANES, comp, 0)

        pltpu.sync_copy(out_v, out_hbm.at[pl.ds(base, b_per_w)])

    return gmf


def kernel(user, item, u_table, i_table, W, b):
    B = user.shape[0]
    R = u_table.shape[0]
    user1d = user.reshape(B)
    item1d = item.reshape(B)
    wfull = jnp.concatenate([
        jnp.broadcast_to(W.reshape(LATENT, 1), (LATENT, LANES)).reshape(-1),
        jnp.broadcast_to(b.reshape(1), (LANES,)),
    ])
    n_grp = LATENT // SUBL
    ut3 = u_table.T.reshape(n_grp, SUBL, R)
    it3 = i_table.T.reshape(n_grp, SUBL, R)
    uflat, iflat = _build_reformat(R)(ut3, it3)
    out = _build_gather(B, R)(
        user1d, item1d, uflat.reshape(-1), iflat.reshape(-1), wfull)
    return out.reshape(B, 1)
